# baseline (speedup 1.0000x reference)
import functools

import jax
import jax.numpy as jnp
from jax import lax
from jax.experimental import pallas as pl
from jax.experimental.pallas import tpu as pltpu
from jax.experimental.pallas import tpu_sc as plsc

_NUM_CORES = 2
_NUM_SUBCORES = 16
_LANES = 16
_HALF = 16
_CW = 1600
_BB = 1024


@functools.cache
def _make_kernel(num_rows: int, dim: int, seq: int, batch: int):
    assert dim == 2 * _HALF and batch == _NUM_SUBCORES * _BB
    n_cchunks = num_rows // _CW
    assert num_rows % _CW == 0 and seq % 2 == 0
    k_iters = (n_cchunks + _NUM_SUBCORES - 1) // _NUM_SUBCORES
    mesh = plsc.VectorSubcoreMesh(
        core_axis_name="c",
        subcore_axis_name="s",
        num_cores=_NUM_CORES,
        num_subcores=_NUM_SUBCORES,
    )

    @functools.partial(
        pl.kernel,
        out_type=jax.ShapeDtypeStruct((seq, dim, batch), jnp.float32),
        mesh=mesh,
        compiler_params=pltpu.CompilerParams(
            use_tc_tiling_on_sc=False, needs_layout_passes=False
        ),
        scratch_types=[
            pltpu.HBM((num_rows, _HALF), jnp.float32),
            pltpu.HBM((num_rows, _HALF), jnp.float32),
            pltpu.SemaphoreType.DMA,
            pltpu.SemaphoreType.DMA,
        ],
    )
    def fused_kernel(tokt_hbm, wt_hbm, out_hbm, half0, half1, gsem, osem):
        c = lax.axis_index("c")
        t = lax.axis_index("s")

        def phase1(half, cbase, stg, pad):
            def stage_src(k):
                col0 = (t + _NUM_SUBCORES * k) * _CW
                return wt_hbm.at[pl.ds(cbase, _HALF), pl.ds(col0, _CW)]

            def start_stage(k, slot):
                pltpu.async_copy(stage_src(k), stg.at[slot], gsem)

            def wait_stage(k, slot):
                pltpu.make_async_copy(stage_src(k), stg.at[slot], gsem).wait()

            def valid(k):
                return t + _NUM_SUBCORES * k < n_cchunks

            start_stage(0, 0)

            @pl.loop(0, k_iters)
            def _cchunk(k):
                slot = lax.rem(k, 2)

                @pl.when(valid(k))
                def _():
                    wait_stage(k, slot)

                @pl.when(valid(k + 1))
                def _():
                    start_stage(k + 1, 1 - slot)

                @pl.when(valid(k))
                def _():
                    @plsc.parallel_loop(0, _CW, step=_LANES, unroll=8)
                    def _t(k0):
                        ridx = k0 + lax.iota(jnp.int32, _LANES)
                        for d in range(_HALF):
                            cidx = jnp.full((_LANES,), d, jnp.int32)
                            plsc.store_scatter(
                                pad, [ridx, cidx], stg[slot, d, pl.ds(k0, _LANES)]
                            )

                    col0 = (t + _NUM_SUBCORES * k) * _CW
                    pltpu.async_copy(
                        pad.at[:, pl.ds(0, _HALF)],
                        half.at[pl.ds(col0, _CW), :],
                        osem,
                    )
                    pltpu.make_async_copy(
                        pad.at[:, pl.ds(0, _HALF)],
                        half.at[pl.ds(col0, _CW), :],
                        osem,
                    ).wait()

        @pl.when(c == 0)
        def _():
            pl.run_scoped(
                functools.partial(phase1, half0, 0),
                pltpu.VMEM((2, _HALF, _CW), jnp.float32),
                pltpu.VMEM((_CW, _HALF + 1), jnp.float32),
            )

        @pl.when(c == 1)
        def _():
            pl.run_scoped(
                functools.partial(phase1, half1, _HALF),
                pltpu.VMEM((2, _HALF, _CW), jnp.float32),
                pltpu.VMEM((_CW, _HALF + 1), jnp.float32),
            )

        plsc.subcore_barrier()

        def phase2(half, cbase, idx_v, rows_v, trans_v):
            b0 = pl.multiple_of(t * _BB, _BB)
            pltpu.sync_copy(tokt_hbm.at[:, pl.ds(b0, _BB)], idx_v)

            def start_gather(s, slot):
                pltpu.async_copy(half.at[idx_v.at[s]], rows_v.at[slot], gsem)

            def wait_gather(s, slot):
                pltpu.make_async_copy(
                    half.at[idx_v.at[s]], rows_v.at[slot], gsem
                ).wait()

            def out_dst(s):
                return out_hbm.at[s, pl.ds(cbase, _HALF), pl.ds(b0, _BB)]

            def start_write(s, slot):
                pltpu.async_copy(
                    trans_v.at[slot, :, pl.ds(0, _BB)], out_dst(s), osem
                )

            def wait_write(s, slot):
                pltpu.make_async_copy(
                    trans_v.at[slot, :, pl.ds(0, _BB)], out_dst(s), osem
                ).wait()

            def transpose_chunk(slot):
                dst = trans_v.at[slot]

                @plsc.parallel_loop(0, _BB, step=1, unroll=8)
                def _t(r):
                    kidx = jnp.full((_LANES,), r, jnp.int32)
                    cidx = lax.iota(jnp.int32, _LANES)
                    plsc.store_scatter(
                        dst, [cidx, kidx], rows_v[slot, r, pl.ds(0, _HALF)]
                    )

            start_gather(0, 0)

            @pl.loop(0, seq, step=2)
            def _chunk_loop(s0):
                for b in range(2):
                    s = s0 + b
                    wait_gather(s, b)

                    @pl.when(s + 1 < seq)
                    def _():
                        start_gather(s + 1, 1 - b)

                    @pl.when(s >= 1)
                    def _():
                        wait_write(s - 1, 1 - b)

                    transpose_chunk(b)
                    start_write(s, b)

            wait_write(seq - 1, (seq - 1) % 2)

        def phase2_scoped(idx_v, rows_v, trans_v):
            @pl.when(c == 0)
            def _():
                phase2(half0, 0, idx_v, rows_v, trans_v)

            @pl.when(c == 1)
            def _():
                phase2(half1, _HALF, idx_v, rows_v, trans_v)

        pl.run_scoped(
            phase2_scoped,
            pltpu.VMEM((seq, _BB), jnp.int32),
            pltpu.VMEM((2, _BB, _HALF), jnp.float32),
            pltpu.VMEM((2, _HALF, _BB + 1), jnp.float32),
        )

    return fused_kernel


def kernel(token_ids, weight):
    b, s = token_ids.shape
    num_rows, dim = weight.shape
    tokt = token_ids.T.astype(jnp.int32)
    wt = weight.T
    out3 = _make_kernel(num_rows, dim, s, b)(tokt, wt)
    return jnp.transpose(out3, (2, 0, 1))


# speedup vs baseline: 4.4916x; 4.4916x over previous
import functools

import jax
import jax.numpy as jnp
from jax import lax
from jax.experimental import pallas as pl
from jax.experimental.pallas import tpu as pltpu
from jax.experimental.pallas import tpu_sc as plsc

_NUM_CORES = 2
_NUM_SUBCORES = 16
_NUM_WORKERS = _NUM_CORES * _NUM_SUBCORES
_LANES = 16


@functools.cache
def _make_gather(num_rows: int, dim: int, seq: int, batch: int):
    bb = batch // _NUM_WORKERS
    assert bb % 128 == 0 and seq % 2 == 0
    mesh = plsc.VectorSubcoreMesh(
        core_axis_name="c",
        subcore_axis_name="s",
        num_cores=_NUM_CORES,
        num_subcores=_NUM_SUBCORES,
    )

    @functools.partial(
        pl.kernel,
        out_type=jax.ShapeDtypeStruct((seq, dim, batch), jnp.float32),
        mesh=mesh,
        compiler_params=pltpu.CompilerParams(
            use_tc_tiling_on_sc=False, needs_layout_passes=False
        ),
        scratch_types=[
            pltpu.VMEM((seq, bb), jnp.int32),
            pltpu.VMEM((2, bb, dim), jnp.float32),
            pltpu.VMEM((2, dim, bb + 1), jnp.float32),
            pltpu.SemaphoreType.DMA,
            pltpu.SemaphoreType.DMA,
        ],
    )
    def gather_kernel(tokt_hbm, w_hbm, out_hbm, idx_v, rows_v, trans_v,
                      gsem, osem):
        wid = lax.axis_index("s") * _NUM_CORES + lax.axis_index("c")
        b0 = pl.multiple_of(wid * bb, bb)
        pltpu.sync_copy(tokt_hbm.at[:, pl.ds(b0, bb)], idx_v)

        def start_gather(s, slot):
            pltpu.async_copy(w_hbm.at[idx_v.at[s]], rows_v.at[slot], gsem)

        def wait_gather(s, slot):
            pltpu.make_async_copy(
                w_hbm.at[idx_v.at[s]], rows_v.at[slot], gsem
            ).wait()

        def start_write(s, slot):
            pltpu.async_copy(
                trans_v.at[slot, :, pl.ds(0, bb)],
                out_hbm.at[s, :, pl.ds(b0, bb)],
                osem,
            )

        def wait_write(s, slot):
            pltpu.make_async_copy(
                trans_v.at[slot, :, pl.ds(0, bb)],
                out_hbm.at[s, :, pl.ds(b0, bb)],
                osem,
            ).wait()

        def transpose_chunk(slot):
            src = rows_v.at[slot]
            dst = trans_v.at[slot]

            @plsc.parallel_loop(0, bb, step=1, unroll=8)
            def _t(r):
                kidx = jnp.full((_LANES,), r, jnp.int32)
                for h in range(dim // _LANES):
                    cidx = h * _LANES + lax.iota(jnp.int32, _LANES)
                    vals = src[r, pl.ds(h * _LANES, _LANES)]
                    plsc.store_scatter(dst, [cidx, kidx], vals)

        start_gather(0, 0)

        @pl.loop(0, seq, step=2)
        def _chunk_loop(s0):
            for b in range(2):
                s = s0 + b
                wait_gather(s, b)

                @pl.when(s + 1 < seq)
                def _():
                    start_gather(s + 1, 1 - b)

                @pl.when(s >= 1)
                def _():
                    wait_write(s - 1, 1 - b)

                transpose_chunk(b)
                start_write(s, b)

        wait_write(seq - 1, (seq - 1) % 2)

    return gather_kernel


def kernel(token_ids, weight):
    b, s = token_ids.shape
    num_rows, dim = weight.shape
    tokt = token_ids.T.astype(jnp.int32)
    out3 = _make_gather(num_rows, dim, s, b)(tokt, weight)
    return jnp.transpose(out3, (2, 0, 1))
